# baseline (speedup 1.0000x reference)
import jax
import jax.numpy as jnp
from jax import lax
from jax.experimental import pallas as pl
from jax.experimental.pallas import tpu as pltpu

N = 4096
D_IN = 128
D_OUT = 128
B = 2048
BQ = 1024
NH = 4
HQ = BQ // NH


def _body(idx_sref, x_ref, w_ref, b_ref, ppr_any, out_ref, enc_ref, buf_ref, sem):
    i = pl.program_id(0)
    nsteps = pl.num_programs(0)

    def issue(block, slot):
        for h in range(NH):

            def one(r, _, h=h):
                row = idx_sref[block * BQ + h * HQ + r]
                pltpu.make_async_copy(
                    ppr_any.at[row], buf_ref.at[slot, h * HQ + r], sem.at[slot, h]
                ).start()
                return 0

            lax.fori_loop(0, HQ, one, 0, unroll=8)

    @pl.when(i == 0)
    def _():
        issue(0, 0)
        enc_ref[...] = (
            jnp.dot(x_ref[...], w_ref[...], preferred_element_type=jnp.float32)
            + b_ref[...]
        )

    @pl.when(i + 1 < nsteps)
    def _():
        issue(i + 1, (i + 1) % 2)

    slot = i % 2
    for h in range(NH):
        pltpu.make_async_copy(
            ppr_any.at[pl.ds(0, HQ)],
            buf_ref.at[slot, pl.ds(h * HQ, HQ)],
            sem.at[slot, h],
        ).wait()
        out_ref[pl.ds(h * HQ, HQ), :] = jnp.dot(
            buf_ref[slot, pl.ds(h * HQ, HQ), :],
            enc_ref[...],
            preferred_element_type=jnp.float32,
        )


def kernel(X, idx, ppr, W, b):
    grid_spec = pltpu.PrefetchScalarGridSpec(
        num_scalar_prefetch=1,
        grid=(B // BQ,),
        in_specs=[
            pl.BlockSpec((N, D_IN), lambda i, idx_ref: (0, 0)),
            pl.BlockSpec((D_IN, D_OUT), lambda i, idx_ref: (0, 0)),
            pl.BlockSpec((1, D_OUT), lambda i, idx_ref: (0, 0)),
            pl.BlockSpec(memory_space=pl.ANY),
        ],
        out_specs=pl.BlockSpec((BQ, D_OUT), lambda i, idx_ref: (i, 0)),
        scratch_shapes=[
            pltpu.VMEM((N, D_OUT), jnp.float32),
            pltpu.VMEM((2, BQ, N), jnp.float32),
            pltpu.SemaphoreType.DMA((2, NH)),
        ],
    )
    return pl.pallas_call(
        _body,
        grid_spec=grid_spec,
        out_shape=jax.ShapeDtypeStruct((B, D_OUT), jnp.float32),
    )(idx.astype(jnp.int32), X, W, b.reshape(1, D_OUT), ppr)


# speedup vs baseline: 1.1189x; 1.1189x over previous
import jax
import jax.numpy as jnp
from jax import lax
from jax.experimental import pallas as pl
from jax.experimental.pallas import tpu as pltpu

N = 4096
D_IN = 128
D_OUT = 128
B = 2048
BQ = 1024
HQ = BQ // 2


def _body(idx_sref, x_ref, w_ref, b_ref, ppr_any, out_ref, enc_ref, buf_ref, sem):
    i = pl.program_id(0)
    nsteps = pl.num_programs(0)

    def issue(block, slot, lo, n, part):
        def one(r, _):
            row = idx_sref[block * BQ + lo + r]
            pltpu.make_async_copy(
                ppr_any.at[row], buf_ref.at[slot, lo + r], sem.at[slot, part]
            ).start()
            return 0

        lax.fori_loop(0, n, one, 0, unroll=8)

    @pl.when(i == 0)
    def _():
        issue(0, 0, 0, BQ, 0)
        enc_ref[...] = (
            jnp.dot(x_ref[...], w_ref[...], preferred_element_type=jnp.float32)
            + b_ref[...]
        )

    @pl.when(i + 1 < nsteps)
    def _():
        nxt = (i + 1) % 2
        issue(i + 1, nxt, 0, HQ, 0)
        issue(i + 1, nxt, HQ, HQ, 1)

    slot = i % 2

    @pl.when(i == 0)
    def _():
        pltpu.make_async_copy(
            ppr_any.at[pl.ds(0, BQ)], buf_ref.at[slot], sem.at[slot, 0]
        ).wait()
        out_ref[...] = jnp.dot(
            buf_ref[slot], enc_ref[...], preferred_element_type=jnp.float32
        )

    @pl.when(i > 0)
    def _():
        for h in range(2):
            pltpu.make_async_copy(
                ppr_any.at[pl.ds(0, HQ)],
                buf_ref.at[slot, pl.ds(h * HQ, HQ)],
                sem.at[slot, h],
            ).wait()
            out_ref[pl.ds(h * HQ, HQ), :] = jnp.dot(
                buf_ref[slot, pl.ds(h * HQ, HQ), :],
                enc_ref[...],
                preferred_element_type=jnp.float32,
            )


def kernel(X, idx, ppr, W, b):
    grid_spec = pltpu.PrefetchScalarGridSpec(
        num_scalar_prefetch=1,
        grid=(B // BQ,),
        in_specs=[
            pl.BlockSpec((N, D_IN), lambda i, idx_ref: (0, 0)),
            pl.BlockSpec((D_IN, D_OUT), lambda i, idx_ref: (0, 0)),
            pl.BlockSpec((1, D_OUT), lambda i, idx_ref: (0, 0)),
            pl.BlockSpec(memory_space=pl.ANY),
        ],
        out_specs=pl.BlockSpec((BQ, D_OUT), lambda i, idx_ref: (i, 0)),
        scratch_shapes=[
            pltpu.VMEM((N, D_OUT), jnp.float32),
            pltpu.VMEM((2, BQ, N), jnp.float32),
            pltpu.SemaphoreType.DMA((2, 2)),
        ],
    )
    return pl.pallas_call(
        _body,
        grid_spec=grid_spec,
        out_shape=jax.ShapeDtypeStruct((B, D_OUT), jnp.float32),
    )(idx.astype(jnp.int32), X, W, b.reshape(1, D_OUT), ppr)


# speedup vs baseline: 1.2882x; 1.1513x over previous
import jax
import jax.numpy as jnp
from jax import lax
from jax.experimental import pallas as pl
from jax.experimental.pallas import tpu as pltpu

N = 4096
D_IN = 128
D_OUT = 128
B = 2048
BQ = 1024
UNROLL = 16


def _body(idx_sref, x_ref, w_ref, b_ref, ppr_any, out_ref, enc_ref, buf_ref, sem):
    i = pl.program_id(0)
    nsteps = pl.num_programs(0)

    def issue(block, slot):
        def one(r, _):
            row = idx_sref[block * BQ + r]
            pltpu.make_async_copy(
                ppr_any.at[row], buf_ref.at[slot, r], sem.at[slot]
            ).start()
            return 0

        lax.fori_loop(0, BQ, one, 0, unroll=UNROLL)

    @pl.when(i == 0)
    def _():
        issue(0, 0)
        enc_ref[...] = (
            jnp.dot(x_ref[...], w_ref[...], preferred_element_type=jnp.float32)
            + b_ref[...]
        )

    @pl.when(i + 1 < nsteps)
    def _():
        issue(i + 1, (i + 1) % 2)

    slot = i % 2
    pltpu.make_async_copy(
        ppr_any.at[pl.ds(0, BQ)], buf_ref.at[slot], sem.at[slot]
    ).wait()
    out_ref[...] = jnp.dot(
        buf_ref[slot], enc_ref[...], preferred_element_type=jnp.float32
    )


def kernel(X, idx, ppr, W, b):
    grid_spec = pltpu.PrefetchScalarGridSpec(
        num_scalar_prefetch=1,
        grid=(B // BQ,),
        in_specs=[
            pl.BlockSpec((N, D_IN), lambda i, idx_ref: (0, 0)),
            pl.BlockSpec((D_IN, D_OUT), lambda i, idx_ref: (0, 0)),
            pl.BlockSpec((1, D_OUT), lambda i, idx_ref: (0, 0)),
            pl.BlockSpec(memory_space=pl.ANY),
        ],
        out_specs=pl.BlockSpec((BQ, D_OUT), lambda i, idx_ref: (i, 0)),
        scratch_shapes=[
            pltpu.VMEM((N, D_OUT), jnp.float32),
            pltpu.VMEM((2, BQ, N), jnp.float32),
            pltpu.SemaphoreType.DMA((2,)),
        ],
    )
    return pl.pallas_call(
        _body,
        grid_spec=grid_spec,
        out_shape=jax.ShapeDtypeStruct((B, D_OUT), jnp.float32),
    )(idx.astype(jnp.int32), X, W, b.reshape(1, D_OUT), ppr)


# speedup vs baseline: 1.3067x; 1.0143x over previous
import jax
import jax.numpy as jnp
from jax import lax
from jax.experimental import pallas as pl
from jax.experimental.pallas import tpu as pltpu

N = 4096
D_IN = 128
D_OUT = 128
B = 2048
BQ = 1024
UNROLL = 32


def _body(idx_sref, x_ref, w_ref, b_ref, ppr_any, out_ref, enc_ref, buf_ref, sem):
    i = pl.program_id(0)
    nsteps = pl.num_programs(0)

    def issue(block, slot):
        def one(r, _):
            row = idx_sref[block * BQ + r]
            pltpu.make_async_copy(
                ppr_any.at[row], buf_ref.at[slot, r], sem.at[slot]
            ).start()
            return 0

        lax.fori_loop(0, BQ, one, 0, unroll=UNROLL)

    @pl.when(i == 0)
    def _():
        issue(0, 0)
        enc_ref[...] = (
            jnp.dot(x_ref[...], w_ref[...], preferred_element_type=jnp.float32)
            + b_ref[...]
        )

    @pl.when(i + 1 < nsteps)
    def _():
        issue(i + 1, (i + 1) % 2)

    slot = i % 2
    pltpu.make_async_copy(
        ppr_any.at[pl.ds(0, BQ)], buf_ref.at[slot], sem.at[slot]
    ).wait()
    out_ref[...] = jnp.dot(
        buf_ref[slot], enc_ref[...], preferred_element_type=jnp.float32
    )


def kernel(X, idx, ppr, W, b):
    grid_spec = pltpu.PrefetchScalarGridSpec(
        num_scalar_prefetch=1,
        grid=(B // BQ,),
        in_specs=[
            pl.BlockSpec((N, D_IN), lambda i, idx_ref: (0, 0)),
            pl.BlockSpec((D_IN, D_OUT), lambda i, idx_ref: (0, 0)),
            pl.BlockSpec((1, D_OUT), lambda i, idx_ref: (0, 0)),
            pl.BlockSpec(memory_space=pl.ANY),
        ],
        out_specs=pl.BlockSpec((BQ, D_OUT), lambda i, idx_ref: (i, 0)),
        scratch_shapes=[
            pltpu.VMEM((N, D_OUT), jnp.float32),
            pltpu.VMEM((2, BQ, N), jnp.float32),
            pltpu.SemaphoreType.DMA((2,)),
        ],
    )
    return pl.pallas_call(
        _body,
        grid_spec=grid_spec,
        out_shape=jax.ShapeDtypeStruct((B, D_OUT), jnp.float32),
    )(idx.astype(jnp.int32), X, W, b.reshape(1, D_OUT), ppr)


# speedup vs baseline: 1.3279x; 1.0162x over previous
import jax
import jax.numpy as jnp
from jax import lax
from jax.experimental import pallas as pl
from jax.experimental.pallas import tpu as pltpu

N = 4096
D_IN = 128
D_OUT = 128
B = 2048
BQ = 1024
UNROLL = 64


def _body(idx_sref, x_ref, w_ref, b_ref, ppr_any, out_ref, enc_ref, buf_ref, sem):
    i = pl.program_id(0)
    nsteps = pl.num_programs(0)

    def issue(block, slot):
        def one(r, _):
            row = idx_sref[block * BQ + r]
            pltpu.make_async_copy(
                ppr_any.at[row], buf_ref.at[slot, r], sem.at[slot]
            ).start()
            return 0

        lax.fori_loop(0, BQ, one, 0, unroll=UNROLL)

    @pl.when(i == 0)
    def _():
        issue(0, 0)
        enc_ref[...] = (
            jnp.dot(x_ref[...], w_ref[...], preferred_element_type=jnp.float32)
            + b_ref[...]
        )

    @pl.when(i + 1 < nsteps)
    def _():
        issue(i + 1, (i + 1) % 2)

    slot = i % 2
    pltpu.make_async_copy(
        ppr_any.at[pl.ds(0, BQ)], buf_ref.at[slot], sem.at[slot]
    ).wait()
    out_ref[...] = jnp.dot(
        buf_ref[slot], enc_ref[...], preferred_element_type=jnp.float32
    )


def kernel(X, idx, ppr, W, b):
    grid_spec = pltpu.PrefetchScalarGridSpec(
        num_scalar_prefetch=1,
        grid=(B // BQ,),
        in_specs=[
            pl.BlockSpec((N, D_IN), lambda i, idx_ref: (0, 0)),
            pl.BlockSpec((D_IN, D_OUT), lambda i, idx_ref: (0, 0)),
            pl.BlockSpec((1, D_OUT), lambda i, idx_ref: (0, 0)),
            pl.BlockSpec(memory_space=pl.ANY),
        ],
        out_specs=pl.BlockSpec((BQ, D_OUT), lambda i, idx_ref: (i, 0)),
        scratch_shapes=[
            pltpu.VMEM((N, D_OUT), jnp.float32),
            pltpu.VMEM((2, BQ, N), jnp.float32),
            pltpu.SemaphoreType.DMA((2,)),
        ],
    )
    return pl.pallas_call(
        _body,
        grid_spec=grid_spec,
        out_shape=jax.ShapeDtypeStruct((B, D_OUT), jnp.float32),
    )(idx.astype(jnp.int32), X, W, b.reshape(1, D_OUT), ppr)
